# Initial kernel scaffold; baseline (speedup 1.0000x reference)
import jax
import jax.numpy as jnp
from jax.experimental import pallas as pl


def kernel(x, senders, receivers, edge_attr, We1, be1, We2, be2, We3, be3, lne_g, lne_b, Wn1, bn1, Wn2, bn2, Wn3, bn3, lnn_g, lnn_b):
    raise NotImplementedError("write your pallas kernel here")


# speedup vs baseline: 3.0216x; 3.0216x over previous
import functools

import jax
import jax.numpy as jnp
from jax import lax
from jax.experimental import pallas as pl
from jax.experimental.pallas import tpu as pltpu
from jax.experimental.pallas import tpu_sc as plsc

N = 10000
E = 320000
D = 128

NC = 2
NS = 16
NW = NC * NS

CH = 128
GROWS = 2 * E // NW
GFULL = GROWS // CH
GTAIL = GROWS - GFULL * CH
SROWS = E // NS
NPT = 632
NPT_LAST = N - 15 * NPT

_sc_mesh = plsc.VectorSubcoreMesh(core_axis_name="c", subcore_axis_name="s")


@functools.partial(
    pl.kernel,
    out_type=jax.ShapeDtypeStruct((2 * E, D), jnp.float32),
    mesh=_sc_mesh,
    scratch_types=[
        pltpu.VMEM((CH,), jnp.int32),
        pltpu.VMEM((GTAIL,), jnp.int32),
        pltpu.VMEM((CH, D), jnp.float32),
        pltpu.SemaphoreType.DMA,
    ],
)
def _sc_gather(x_hbm, idx_hbm, out_hbm, idx_v, idx_t, rows_v, sem):
    c = lax.axis_index("c")
    s = lax.axis_index("s")
    wid = s * NC + c
    base_w = wid * GROWS

    def body(j, carry):
        base = base_w + j * CH
        pltpu.sync_copy(idx_hbm.at[pl.ds(base, CH)], idx_v)
        pltpu.async_copy(x_hbm.at[idx_v], rows_v, sem).wait()
        pltpu.sync_copy(rows_v, out_hbm.at[pl.ds(base, CH)])
        return carry

    lax.fori_loop(0, GFULL, body, 0)

    base = base_w + GFULL * CH
    pltpu.sync_copy(idx_hbm.at[pl.ds(base, GTAIL)], idx_t)
    pltpu.async_copy(x_hbm.at[idx_t], rows_v.at[pl.ds(0, GTAIL)], sem).wait()
    pltpu.sync_copy(rows_v.at[pl.ds(0, GTAIL)], out_hbm.at[pl.ds(base, GTAIL)])


@functools.partial(
    pl.kernel,
    out_type=jax.ShapeDtypeStruct((2 * N, D), jnp.float32),
    mesh=_sc_mesh,
    scratch_types=[
        pltpu.VMEM((CH,), jnp.int32),
        pltpu.VMEM((GTAIL,), jnp.int32),
        pltpu.VMEM((CH, D), jnp.float32),
        pltpu.VMEM_SHARED((N, D), jnp.float32),
        pltpu.SemaphoreType.DMA,
    ],
)
def _sc_scatter(msg_hbm, idx_hbm, zero_hbm, out_hbm, idx_v, idx_t, rows_v, acc, sem):
    c = lax.axis_index("c")
    s = lax.axis_index("s")

    abase = s * NPT
    pltpu.sync_copy(zero_hbm.at[pl.ds(0, NPT_LAST)], acc.at[pl.ds(abase, NPT_LAST)])

    @pl.when(s < NS - 1)
    def _():
        pltpu.sync_copy(zero_hbm.at[pl.ds(0, NPT - NPT_LAST)],
                        acc.at[pl.ds(abase + NPT_LAST, NPT - NPT_LAST)])

    plsc.subcore_barrier()

    ebase = s * SROWS

    def body(j, carry):
        b = ebase + j * CH
        pltpu.sync_copy(idx_hbm.at[pl.ds(c * E + b, CH)], idx_v)
        pltpu.sync_copy(msg_hbm.at[pl.ds(b, CH)], rows_v)
        pltpu.sync_copy(rows_v, acc.at[idx_v], add=True)
        return carry

    lax.fori_loop(0, GFULL, body, 0)

    b = ebase + GFULL * CH
    pltpu.sync_copy(idx_hbm.at[pl.ds(c * E + b, GTAIL)], idx_t)
    pltpu.sync_copy(msg_hbm.at[pl.ds(b, GTAIL)], rows_v.at[pl.ds(0, GTAIL)])
    pltpu.sync_copy(rows_v.at[pl.ds(0, GTAIL)], acc.at[idx_t], add=True)
    plsc.subcore_barrier()

    pltpu.sync_copy(acc.at[pl.ds(abase, NPT_LAST)],
                    out_hbm.at[pl.ds(c * N + abase, NPT_LAST)])

    @pl.when(s < NS - 1)
    def _():
        pltpu.sync_copy(acc.at[pl.ds(abase + NPT_LAST, NPT - NPT_LAST)],
                        out_hbm.at[pl.ds(c * N + abase + NPT_LAST, NPT - NPT_LAST)])


def _edge_body(ea_ref, xr_ref, xs_ref, w1_ref, b1_ref, w2_ref, b2_ref,
               w3_ref, b3_ref, g_ref, b_ref, msg_ref, eout_ref):
    ea = ea_ref[...]
    f32 = jnp.float32
    h = jnp.dot(ea, w1_ref[0:D, :], preferred_element_type=f32)
    h += jnp.dot(xr_ref[...], w1_ref[D:2 * D, :], preferred_element_type=f32)
    h += jnp.dot(xs_ref[...], w1_ref[2 * D:3 * D, :], preferred_element_type=f32)
    h = jnp.tanh(h + b1_ref[...])
    h = jnp.tanh(jnp.dot(h, w2_ref[...], preferred_element_type=f32) + b2_ref[...])
    h = jnp.dot(h, w3_ref[...], preferred_element_type=f32) + b3_ref[...]
    mu = jnp.mean(h, axis=-1, keepdims=True)
    hc = h - mu
    var = jnp.mean(hc * hc, axis=-1, keepdims=True)
    m = hc * lax.rsqrt(var + 1e-5) * g_ref[...] + b_ref[...]
    msg_ref[...] = m
    eout_ref[...] = ea + m


def _edge_mlp(edge_attr, xg, We1, be1, We2, be2, We3, be3, g, b, block):
    nb = E // block
    row = lambda i: (i, 0)
    full = lambda shape: pl.BlockSpec(shape, lambda i: (0, 0))
    return pl.pallas_call(
        _edge_body,
        grid=(nb,),
        in_specs=[
            pl.BlockSpec((block, D), row),
            pl.BlockSpec((block, D), row),
            pl.BlockSpec((block, D), lambda i: (i + nb, 0)),
            full((3 * D, D)), full((1, D)),
            full((D, D)), full((1, D)),
            full((D, D)), full((1, D)),
            full((1, D)), full((1, D)),
        ],
        out_specs=[pl.BlockSpec((block, D), row), pl.BlockSpec((block, D), row)],
        out_shape=[
            jax.ShapeDtypeStruct((E, D), jnp.float32),
            jax.ShapeDtypeStruct((E, D), jnp.float32),
        ],
    )(edge_attr, xg, xg, We1, be1, We2, be2, We3, be3, g, b)


def _node_body(x_ref, ar_ref, as_ref, w1_ref, b1_ref, w2_ref, b2_ref,
               w3_ref, b3_ref, g_ref, b_ref, out_ref):
    x = x_ref[...]
    agg = ar_ref[...] - as_ref[...]
    f32 = jnp.float32
    h = jnp.dot(x, w1_ref[0:D, :], preferred_element_type=f32)
    h += jnp.dot(agg, w1_ref[D:2 * D, :], preferred_element_type=f32)
    h = jnp.tanh(h + b1_ref[...])
    h = jnp.tanh(jnp.dot(h, w2_ref[...], preferred_element_type=f32) + b2_ref[...])
    h = jnp.dot(h, w3_ref[...], preferred_element_type=f32) + b3_ref[...]
    mu = jnp.mean(h, axis=-1, keepdims=True)
    hc = h - mu
    var = jnp.mean(hc * hc, axis=-1, keepdims=True)
    out_ref[...] = hc * lax.rsqrt(var + 1e-5) * g_ref[...] + b_ref[...] + x


def _node_mlp(x, agg2, Wn1, bn1, Wn2, bn2, Wn3, bn3, g, b, block):
    nb = N // block
    row = lambda i: (i, 0)
    full = lambda shape: pl.BlockSpec(shape, lambda i: (0, 0))
    return pl.pallas_call(
        _node_body,
        grid=(nb,),
        in_specs=[
            pl.BlockSpec((block, D), row),
            pl.BlockSpec((block, D), row),
            pl.BlockSpec((block, D), lambda i: (i + nb, 0)),
            full((2 * D, D)), full((1, D)),
            full((D, D)), full((1, D)),
            full((D, D)), full((1, D)),
            full((1, D)), full((1, D)),
        ],
        out_specs=pl.BlockSpec((block, D), row),
        out_shape=jax.ShapeDtypeStruct((N, D), jnp.float32),
    )(x, agg2, agg2, Wn1, bn1, Wn2, bn2, Wn3, bn3, g, b)


@jax.jit
def kernel(x, senders, receivers, edge_attr,
           We1, be1, We2, be2, We3, be3, lne_g, lne_b,
           Wn1, bn1, Wn2, bn2, Wn3, bn3, lnn_g, lnn_b):
    idx = jnp.concatenate([receivers, senders])

    xg = _sc_gather(x, idx)

    r2 = lambda v: v.reshape(1, D)
    messages, edge_out = _edge_mlp(
        edge_attr, xg, We1, r2(be1), We2, r2(be2), We3, r2(be3),
        r2(lne_g), r2(lne_b), block=1280)

    zeros_nd = jnp.zeros((NPT_LAST, D), jnp.float32)
    agg2 = _sc_scatter(messages, idx, zeros_nd)

    x_out = _node_mlp(x, agg2, Wn1, r2(bn1), Wn2, r2(bn2), Wn3, r2(bn3),
                      r2(lnn_g), r2(lnn_b), block=1000)

    return x_out, edge_out


# speedup vs baseline: 4.0776x; 1.3495x over previous
import functools

import jax
import jax.numpy as jnp
from jax import lax
from jax.experimental import pallas as pl
from jax.experimental.pallas import tpu as pltpu
from jax.experimental.pallas import tpu_sc as plsc

N = 10000
E = 320000
D = 128

NC = 2
NS = 16
NW = NC * NS

CH = 128
GROWS = 2 * E // NW
GFULL = GROWS // CH
GTAIL = GROWS - GFULL * CH
SROWS = E // NS
NPT = 632
NPT_LAST = N - 15 * NPT

_sc_mesh = plsc.VectorSubcoreMesh(core_axis_name="c", subcore_axis_name="s")


@functools.partial(
    pl.kernel,
    out_type=jax.ShapeDtypeStruct((2 * E, D), jnp.float32),
    mesh=_sc_mesh,
    scratch_types=[
        pltpu.VMEM((CH,), jnp.int32),
        pltpu.VMEM((CH,), jnp.int32),
        pltpu.VMEM((GTAIL,), jnp.int32),
        pltpu.VMEM((CH, D), jnp.float32),
        pltpu.VMEM((CH, D), jnp.float32),
        pltpu.VMEM((GTAIL, D), jnp.float32),
        pltpu.SemaphoreType.DMA,
        pltpu.SemaphoreType.DMA,
        pltpu.SemaphoreType.DMA,
        pltpu.SemaphoreType.DMA,
        pltpu.SemaphoreType.DMA,
        pltpu.SemaphoreType.DMA,
        pltpu.SemaphoreType.DMA,
    ],
)
def _sc_gather(x_hbm, idx_hbm, out_hbm, idx0, idx1, idxt, r0, r1, rt,
               si0, si1, sg0, sg1, ss0, ss1, st):
    c = lax.axis_index("c")
    s = lax.axis_index("s")
    wid = s * NC + c
    base_w = wid * GROWS
    idxb, rows = (idx0, idx1), (r0, r1)
    si, sg, ss = (si0, si1), (sg0, sg1), (ss0, ss1)

    def start_idx(j, b):
        pltpu.async_copy(idx_hbm.at[pl.ds(base_w + j * CH, CH)], idxb[b], si[b])

    def wait_idx(b):
        pltpu.make_async_copy(idx_hbm.at[pl.ds(base_w, CH)], idxb[b], si[b]).wait()

    def start_gather(b):
        pltpu.async_copy(x_hbm.at[idxb[b]], rows[b], sg[b])

    def wait_gather(b):
        pltpu.make_async_copy(x_hbm.at[idxb[b]], rows[b], sg[b]).wait()

    def start_store(j, b):
        pltpu.async_copy(rows[b], out_hbm.at[pl.ds(base_w + j * CH, CH)], ss[b])

    def wait_store(b):
        pltpu.make_async_copy(rows[b], out_hbm.at[pl.ds(base_w, CH)], ss[b]).wait()

    def chunk(j, b, wait_prev_store):
        nb = 1 - b
        start_idx(j + 1, nb)
        wait_gather(b)
        start_store(j, b)
        wait_idx(nb)
        if wait_prev_store:
            wait_store(nb)
        start_gather(nb)

    pltpu.sync_copy(idx_hbm.at[pl.ds(base_w, CH)], idx0)
    start_gather(0)
    chunk(0, 0, wait_prev_store=False)
    chunk(1, 1, wait_prev_store=True)

    @pl.loop(2, GFULL - 2, step=2)
    def _(j0):
        chunk(j0, 0, wait_prev_store=True)
        chunk(j0 + 1, 1, wait_prev_store=True)

    chunk(GFULL - 2, 0, wait_prev_store=True)
    wait_gather(1)
    start_store(GFULL - 1, 1)

    tb = base_w + GFULL * CH
    pltpu.sync_copy(idx_hbm.at[pl.ds(tb, GTAIL)], idxt)
    pltpu.async_copy(x_hbm.at[idxt], rt, st).wait()
    pltpu.sync_copy(rt, out_hbm.at[pl.ds(tb, GTAIL)])

    wait_store(0)
    wait_store(1)


@functools.partial(
    pl.kernel,
    out_type=jax.ShapeDtypeStruct((2 * N, D), jnp.float32),
    mesh=_sc_mesh,
    scratch_types=[
        pltpu.VMEM((CH,), jnp.int32),
        pltpu.VMEM((CH,), jnp.int32),
        pltpu.VMEM((GTAIL,), jnp.int32),
        pltpu.VMEM((CH, D), jnp.float32),
        pltpu.VMEM((CH, D), jnp.float32),
        pltpu.VMEM((GTAIL, D), jnp.float32),
        pltpu.VMEM_SHARED((N, D), jnp.float32),
        pltpu.SemaphoreType.DMA,
        pltpu.SemaphoreType.DMA,
        pltpu.SemaphoreType.DMA,
        pltpu.SemaphoreType.DMA,
        pltpu.SemaphoreType.DMA,
        pltpu.SemaphoreType.DMA,
    ],
)
def _sc_scatter(msg_hbm, idx_hbm, zero_hbm, out_hbm, idx0, idx1, idxt,
                r0, r1, rt, acc, si0, si1, sm0, sm1, sa0, sa1):
    c = lax.axis_index("c")
    s = lax.axis_index("s")
    idxb, rows = (idx0, idx1), (r0, r1)
    si, sm, sa = (si0, si1), (sm0, sm1), (sa0, sa1)
    ebase = s * SROWS

    def start_loads(j, b):
        pltpu.async_copy(idx_hbm.at[pl.ds(c * E + ebase + j * CH, CH)], idxb[b], si[b])
        pltpu.async_copy(msg_hbm.at[pl.ds(ebase + j * CH, CH)], rows[b], sm[b])

    def wait_loads(b):
        pltpu.make_async_copy(idx_hbm.at[pl.ds(ebase, CH)], idxb[b], si[b]).wait()
        pltpu.make_async_copy(msg_hbm.at[pl.ds(ebase, CH)], rows[b], sm[b]).wait()

    def start_scatter(b):
        pltpu.async_copy(rows[b], acc.at[idxb[b]], sa[b], add=True)

    def wait_scatter(b):
        pltpu.make_async_copy(rows[b], acc.at[idxb[b]], sa[b]).wait()

    start_loads(0, 0)

    abase = s * NPT
    pltpu.sync_copy(zero_hbm.at[pl.ds(0, NPT_LAST)], acc.at[pl.ds(abase, NPT_LAST)])

    @pl.when(s < NS - 1)
    def _():
        pltpu.sync_copy(zero_hbm.at[pl.ds(0, NPT - NPT_LAST)],
                        acc.at[pl.ds(abase + NPT_LAST, NPT - NPT_LAST)])

    plsc.subcore_barrier()

    def chunk(j, b, wait_prev_scatter, start_next):
        nb = 1 - b
        if wait_prev_scatter:
            wait_scatter(nb)
        if start_next:
            start_loads(j + 1, nb)
        wait_loads(b)
        start_scatter(b)

    chunk(0, 0, wait_prev_scatter=False, start_next=True)

    @pl.loop(1, GFULL - 1, step=2)
    def _(j0):
        chunk(j0, 1, wait_prev_scatter=True, start_next=True)
        chunk(j0 + 1, 0, wait_prev_scatter=True, start_next=True)

    chunk(GFULL - 1, 1, wait_prev_scatter=True, start_next=False)
    wait_scatter(1)

    b = ebase + GFULL * CH
    pltpu.sync_copy(idx_hbm.at[pl.ds(c * E + b, GTAIL)], idxt)
    pltpu.sync_copy(msg_hbm.at[pl.ds(b, GTAIL)], rt)
    pltpu.sync_copy(rt, acc.at[idxt], add=True)
    plsc.subcore_barrier()

    pltpu.sync_copy(acc.at[pl.ds(abase, NPT_LAST)],
                    out_hbm.at[pl.ds(c * N + abase, NPT_LAST)])

    @pl.when(s < NS - 1)
    def _():
        pltpu.sync_copy(acc.at[pl.ds(abase + NPT_LAST, NPT - NPT_LAST)],
                        out_hbm.at[pl.ds(c * N + abase + NPT_LAST, NPT - NPT_LAST)])


def _edge_body(ea_ref, xr_ref, xs_ref, w1_ref, b1_ref, w2_ref, b2_ref,
               w3_ref, b3_ref, g_ref, b_ref, msg_ref, eout_ref):
    ea = ea_ref[...]
    f32 = jnp.float32
    h = jnp.dot(ea, w1_ref[0:D, :], preferred_element_type=f32)
    h += jnp.dot(xr_ref[...], w1_ref[D:2 * D, :], preferred_element_type=f32)
    h += jnp.dot(xs_ref[...], w1_ref[2 * D:3 * D, :], preferred_element_type=f32)
    h = jnp.tanh(h + b1_ref[...])
    h = jnp.tanh(jnp.dot(h, w2_ref[...], preferred_element_type=f32) + b2_ref[...])
    h = jnp.dot(h, w3_ref[...], preferred_element_type=f32) + b3_ref[...]
    mu = jnp.mean(h, axis=-1, keepdims=True)
    hc = h - mu
    var = jnp.mean(hc * hc, axis=-1, keepdims=True)
    m = hc * lax.rsqrt(var + 1e-5) * g_ref[...] + b_ref[...]
    msg_ref[...] = m
    eout_ref[...] = ea + m


def _edge_mlp(edge_attr, xg, We1, be1, We2, be2, We3, be3, g, b, block):
    nb = E // block
    row = lambda i: (i, 0)
    full = lambda shape: pl.BlockSpec(shape, lambda i: (0, 0))
    return pl.pallas_call(
        _edge_body,
        grid=(nb,),
        in_specs=[
            pl.BlockSpec((block, D), row),
            pl.BlockSpec((block, D), row),
            pl.BlockSpec((block, D), lambda i: (i + nb, 0)),
            full((3 * D, D)), full((1, D)),
            full((D, D)), full((1, D)),
            full((D, D)), full((1, D)),
            full((1, D)), full((1, D)),
        ],
        out_specs=[pl.BlockSpec((block, D), row), pl.BlockSpec((block, D), row)],
        out_shape=[
            jax.ShapeDtypeStruct((E, D), jnp.float32),
            jax.ShapeDtypeStruct((E, D), jnp.float32),
        ],
    )(edge_attr, xg, xg, We1, be1, We2, be2, We3, be3, g, b)


def _node_body(x_ref, ar_ref, as_ref, w1_ref, b1_ref, w2_ref, b2_ref,
               w3_ref, b3_ref, g_ref, b_ref, out_ref):
    x = x_ref[...]
    agg = ar_ref[...] - as_ref[...]
    f32 = jnp.float32
    h = jnp.dot(x, w1_ref[0:D, :], preferred_element_type=f32)
    h += jnp.dot(agg, w1_ref[D:2 * D, :], preferred_element_type=f32)
    h = jnp.tanh(h + b1_ref[...])
    h = jnp.tanh(jnp.dot(h, w2_ref[...], preferred_element_type=f32) + b2_ref[...])
    h = jnp.dot(h, w3_ref[...], preferred_element_type=f32) + b3_ref[...]
    mu = jnp.mean(h, axis=-1, keepdims=True)
    hc = h - mu
    var = jnp.mean(hc * hc, axis=-1, keepdims=True)
    out_ref[...] = hc * lax.rsqrt(var + 1e-5) * g_ref[...] + b_ref[...] + x


def _node_mlp(x, agg2, Wn1, bn1, Wn2, bn2, Wn3, bn3, g, b, block):
    nb = N // block
    row = lambda i: (i, 0)
    full = lambda shape: pl.BlockSpec(shape, lambda i: (0, 0))
    return pl.pallas_call(
        _node_body,
        grid=(nb,),
        in_specs=[
            pl.BlockSpec((block, D), row),
            pl.BlockSpec((block, D), row),
            pl.BlockSpec((block, D), lambda i: (i + nb, 0)),
            full((2 * D, D)), full((1, D)),
            full((D, D)), full((1, D)),
            full((D, D)), full((1, D)),
            full((1, D)), full((1, D)),
        ],
        out_specs=pl.BlockSpec((block, D), row),
        out_shape=jax.ShapeDtypeStruct((N, D), jnp.float32),
    )(x, agg2, agg2, Wn1, bn1, Wn2, bn2, Wn3, bn3, g, b)


@jax.jit
def kernel(x, senders, receivers, edge_attr,
           We1, be1, We2, be2, We3, be3, lne_g, lne_b,
           Wn1, bn1, Wn2, bn2, Wn3, bn3, lnn_g, lnn_b):
    idx = jnp.concatenate([receivers, senders])

    xg = _sc_gather(x, idx)

    r2 = lambda v: v.reshape(1, D)
    messages, edge_out = _edge_mlp(
        edge_attr, xg, We1, r2(be1), We2, r2(be2), We3, r2(be3),
        r2(lne_g), r2(lne_b), block=1280)

    zeros_nd = jnp.zeros((NPT_LAST, D), jnp.float32)
    agg2 = _sc_scatter(messages, idx, zeros_nd)

    x_out = _node_mlp(x, agg2, Wn1, r2(bn1), Wn2, r2(bn2), Wn3, r2(bn3),
                      r2(lnn_g), r2(lnn_b), block=1000)

    return x_out, edge_out


# speedup vs baseline: 4.4886x; 1.1008x over previous
import functools

import jax
import jax.numpy as jnp
from jax import lax
from jax.experimental import pallas as pl
from jax.experimental.pallas import tpu as pltpu
from jax.experimental.pallas import tpu_sc as plsc

N = 10000
E = 320000
D = 128

NC = 2
NS = 16
NW = NC * NS

CH = 128

NPT = 632
NPT_LAST = N - 15 * NPT

_sc_mesh = plsc.VectorSubcoreMesh(core_axis_name="c", subcore_axis_name="s")


def _make_sc_gather(nrows):
    grows = nrows // NW
    gfull = grows // CH
    gtail = grows - gfull * CH
    assert nrows % NW == 0 and gtail % 8 == 0 and gtail > 0 and gfull % 2 == 0

    @functools.partial(
        pl.kernel,
        out_type=jax.ShapeDtypeStruct((nrows, D), jnp.float32),
        mesh=_sc_mesh,
        scratch_types=[
            pltpu.VMEM((CH,), jnp.int32),
            pltpu.VMEM((CH,), jnp.int32),
            pltpu.VMEM((gtail,), jnp.int32),
            pltpu.VMEM((CH, D), jnp.float32),
            pltpu.VMEM((CH, D), jnp.float32),
            pltpu.VMEM((gtail, D), jnp.float32),
            pltpu.SemaphoreType.DMA,
            pltpu.SemaphoreType.DMA,
            pltpu.SemaphoreType.DMA,
            pltpu.SemaphoreType.DMA,
            pltpu.SemaphoreType.DMA,
            pltpu.SemaphoreType.DMA,
            pltpu.SemaphoreType.DMA,
        ],
    )
    def sc_gather(x_hbm, idx_hbm, out_hbm, idx0, idx1, idxt, r0, r1, rt,
                  si0, si1, sg0, sg1, ss0, ss1, st):
        c = lax.axis_index("c")
        s = lax.axis_index("s")
        base_w = (s * NC + c) * grows
        idxb, rows = (idx0, idx1), (r0, r1)
        si, sg, ss = (si0, si1), (sg0, sg1), (ss0, ss1)

        def start_idx(j, b):
            pltpu.async_copy(idx_hbm.at[pl.ds(base_w + j * CH, CH)], idxb[b], si[b])

        def wait_idx(b):
            pltpu.make_async_copy(idx_hbm.at[pl.ds(base_w, CH)], idxb[b], si[b]).wait()

        def start_gather(b):
            pltpu.async_copy(x_hbm.at[idxb[b]], rows[b], sg[b])

        def wait_gather(b):
            pltpu.make_async_copy(x_hbm.at[idxb[b]], rows[b], sg[b]).wait()

        def start_store(j, b):
            pltpu.async_copy(rows[b], out_hbm.at[pl.ds(base_w + j * CH, CH)], ss[b])

        def wait_store(b):
            pltpu.make_async_copy(rows[b], out_hbm.at[pl.ds(base_w, CH)], ss[b]).wait()

        def chunk(j, b, wait_prev_store):
            nb = 1 - b
            start_idx(j + 1, nb)
            wait_gather(b)
            start_store(j, b)
            wait_idx(nb)
            if wait_prev_store:
                wait_store(nb)
            start_gather(nb)

        pltpu.sync_copy(idx_hbm.at[pl.ds(base_w, CH)], idx0)
        start_gather(0)
        chunk(0, 0, wait_prev_store=False)
        chunk(1, 1, wait_prev_store=True)

        @pl.loop(2, gfull - 2, step=2)
        def _(j0):
            chunk(j0, 0, wait_prev_store=True)
            chunk(j0 + 1, 1, wait_prev_store=True)

        chunk(gfull - 2, 0, wait_prev_store=True)
        wait_gather(1)
        start_store(gfull - 1, 1)

        tb = base_w + gfull * CH
        pltpu.sync_copy(idx_hbm.at[pl.ds(tb, gtail)], idxt)
        pltpu.async_copy(x_hbm.at[idxt], rt, st).wait()
        pltpu.sync_copy(rt, out_hbm.at[pl.ds(tb, gtail)])

        wait_store(0)
        wait_store(1)

    return sc_gather


def _make_sc_scatter(ne):
    srows = ne // NS
    sfull = srows // CH
    stail = srows - sfull * CH
    assert ne % NS == 0 and stail % 8 == 0 and stail > 0 and sfull % 2 == 0

    @functools.partial(
        pl.kernel,
        out_type=jax.ShapeDtypeStruct((2 * N, D), jnp.float32),
        mesh=_sc_mesh,
        scratch_types=[
            pltpu.VMEM((CH,), jnp.int32),
            pltpu.VMEM((CH,), jnp.int32),
            pltpu.VMEM((stail,), jnp.int32),
            pltpu.VMEM((CH, D), jnp.float32),
            pltpu.VMEM((CH, D), jnp.float32),
            pltpu.VMEM((stail, D), jnp.float32),
            pltpu.VMEM_SHARED((N, D), jnp.float32),
            pltpu.SemaphoreType.DMA,
            pltpu.SemaphoreType.DMA,
            pltpu.SemaphoreType.DMA,
            pltpu.SemaphoreType.DMA,
            pltpu.SemaphoreType.DMA,
            pltpu.SemaphoreType.DMA,
        ],
    )
    def sc_scatter(msg_hbm, idx_hbm, zero_hbm, out_hbm, idx0, idx1, idxt,
                   r0, r1, rt, acc, si0, si1, sm0, sm1, sa0, sa1):
        c = lax.axis_index("c")
        s = lax.axis_index("s")
        idxb, rows = (idx0, idx1), (r0, r1)
        si, sm, sa = (si0, si1), (sm0, sm1), (sa0, sa1)
        ebase = s * srows

        def start_loads(j, b):
            pltpu.async_copy(idx_hbm.at[pl.ds(c * ne + ebase + j * CH, CH)],
                             idxb[b], si[b])
            pltpu.async_copy(msg_hbm.at[pl.ds(ebase + j * CH, CH)], rows[b], sm[b])

        def wait_loads(b):
            pltpu.make_async_copy(idx_hbm.at[pl.ds(ebase, CH)], idxb[b], si[b]).wait()
            pltpu.make_async_copy(msg_hbm.at[pl.ds(ebase, CH)], rows[b], sm[b]).wait()

        def start_scatter(b):
            pltpu.async_copy(rows[b], acc.at[idxb[b]], sa[b], add=True)

        def wait_scatter(b):
            pltpu.make_async_copy(rows[b], acc.at[idxb[b]], sa[b]).wait()

        start_loads(0, 0)

        abase = s * NPT
        pltpu.sync_copy(zero_hbm.at[pl.ds(0, NPT_LAST)],
                        acc.at[pl.ds(abase, NPT_LAST)])

        @pl.when(s < NS - 1)
        def _():
            pltpu.sync_copy(zero_hbm.at[pl.ds(0, NPT - NPT_LAST)],
                            acc.at[pl.ds(abase + NPT_LAST, NPT - NPT_LAST)])

        plsc.subcore_barrier()

        def chunk(j, b, wait_prev_scatter, start_next):
            nb = 1 - b
            if wait_prev_scatter:
                wait_scatter(nb)
            if start_next:
                start_loads(j + 1, nb)
            wait_loads(b)
            start_scatter(b)

        chunk(0, 0, wait_prev_scatter=False, start_next=True)

        @pl.loop(1, sfull - 1, step=2)
        def _(j0):
            chunk(j0, 1, wait_prev_scatter=True, start_next=True)
            chunk(j0 + 1, 0, wait_prev_scatter=True, start_next=True)

        chunk(sfull - 1, 1, wait_prev_scatter=True, start_next=False)
        wait_scatter(1)

        b = ebase + sfull * CH
        pltpu.sync_copy(idx_hbm.at[pl.ds(c * ne + b, stail)], idxt)
        pltpu.sync_copy(msg_hbm.at[pl.ds(b, stail)], rt)
        pltpu.sync_copy(rt, acc.at[idxt], add=True)
        plsc.subcore_barrier()

        pltpu.sync_copy(acc.at[pl.ds(abase, NPT_LAST)],
                        out_hbm.at[pl.ds(c * N + abase, NPT_LAST)])

        @pl.when(s < NS - 1)
        def _():
            pltpu.sync_copy(acc.at[pl.ds(abase + NPT_LAST, NPT - NPT_LAST)],
                            out_hbm.at[pl.ds(c * N + abase + NPT_LAST,
                                             NPT - NPT_LAST)])

    return sc_scatter


def _edge_body(ea_ref, xr_ref, xs_ref, w1_ref, b1_ref, w2_ref, b2_ref,
               w3_ref, b3_ref, g_ref, b_ref, msg_ref, eout_ref):
    ea = ea_ref[...]
    f32 = jnp.float32
    h = jnp.dot(ea, w1_ref[0:D, :], preferred_element_type=f32)
    h += jnp.dot(xr_ref[...], w1_ref[D:2 * D, :], preferred_element_type=f32)
    h += jnp.dot(xs_ref[...], w1_ref[2 * D:3 * D, :], preferred_element_type=f32)
    h = jnp.tanh(h + b1_ref[...])
    h = jnp.tanh(jnp.dot(h, w2_ref[...], preferred_element_type=f32) + b2_ref[...])
    h = jnp.dot(h, w3_ref[...], preferred_element_type=f32) + b3_ref[...]
    mu = jnp.mean(h, axis=-1, keepdims=True)
    hc = h - mu
    var = jnp.mean(hc * hc, axis=-1, keepdims=True)
    m = hc * lax.rsqrt(var + 1e-5) * g_ref[...] + b_ref[...]
    msg_ref[...] = m
    eout_ref[...] = ea + m


def _edge_mlp(edge_attr, xg, We1, be1, We2, be2, We3, be3, g, b, ne, block,
              ea_off):
    nb = ne // block
    row = lambda i: (i, 0)
    full = lambda shape: pl.BlockSpec(shape, lambda i: (0, 0))
    return pl.pallas_call(
        _edge_body,
        grid=(nb,),
        in_specs=[
            pl.BlockSpec((block, D), lambda i: (i + ea_off, 0)),
            pl.BlockSpec((block, D), row),
            pl.BlockSpec((block, D), lambda i: (i + nb, 0)),
            full((3 * D, D)), full((1, D)),
            full((D, D)), full((1, D)),
            full((D, D)), full((1, D)),
            full((1, D)), full((1, D)),
        ],
        out_specs=[pl.BlockSpec((block, D), row), pl.BlockSpec((block, D), row)],
        out_shape=[
            jax.ShapeDtypeStruct((ne, D), jnp.float32),
            jax.ShapeDtypeStruct((ne, D), jnp.float32),
        ],
    )(edge_attr, xg, xg, We1, be1, We2, be2, We3, be3, g, b)


def _node_body(x_ref, ar_ref, as_ref, br_ref, bs_ref, w1_ref, b1_ref,
               w2_ref, b2_ref, w3_ref, b3_ref, g_ref, b_ref, out_ref):
    x = x_ref[...]
    agg = (ar_ref[...] + br_ref[...]) - (as_ref[...] + bs_ref[...])
    f32 = jnp.float32
    h = jnp.dot(x, w1_ref[0:D, :], preferred_element_type=f32)
    h += jnp.dot(agg, w1_ref[D:2 * D, :], preferred_element_type=f32)
    h = jnp.tanh(h + b1_ref[...])
    h = jnp.tanh(jnp.dot(h, w2_ref[...], preferred_element_type=f32) + b2_ref[...])
    h = jnp.dot(h, w3_ref[...], preferred_element_type=f32) + b3_ref[...]
    mu = jnp.mean(h, axis=-1, keepdims=True)
    hc = h - mu
    var = jnp.mean(hc * hc, axis=-1, keepdims=True)
    out_ref[...] = hc * lax.rsqrt(var + 1e-5) * g_ref[...] + b_ref[...] + x


def _node_mlp(x, aggA, aggB, Wn1, bn1, Wn2, bn2, Wn3, bn3, g, b, block):
    nb = N // block
    row = lambda i: (i, 0)
    shift = lambda i: (i + nb, 0)
    full = lambda shape: pl.BlockSpec(shape, lambda i: (0, 0))
    return pl.pallas_call(
        _node_body,
        grid=(nb,),
        in_specs=[
            pl.BlockSpec((block, D), row),
            pl.BlockSpec((block, D), row),
            pl.BlockSpec((block, D), shift),
            pl.BlockSpec((block, D), row),
            pl.BlockSpec((block, D), shift),
            full((2 * D, D)), full((1, D)),
            full((D, D)), full((1, D)),
            full((D, D)), full((1, D)),
            full((1, D)), full((1, D)),
        ],
        out_specs=pl.BlockSpec((block, D), row),
        out_shape=jax.ShapeDtypeStruct((N, D), jnp.float32),
    )(x, aggA, aggA, aggB, aggB, Wn1, bn1, Wn2, bn2, Wn3, bn3, g, b)


EH = E // 2
_sc_gather_h = _make_sc_gather(2 * EH)
_sc_scatter_h = _make_sc_scatter(EH)


@jax.jit
def kernel(x, senders, receivers, edge_attr,
           We1, be1, We2, be2, We3, be3, lne_g, lne_b,
           Wn1, bn1, Wn2, bn2, Wn3, bn3, lnn_g, lnn_b):
    r2 = lambda v: v.reshape(1, D)
    zeros_nd = jnp.zeros((NPT_LAST, D), jnp.float32)

    idxA = jnp.concatenate([receivers[:EH], senders[:EH]])
    idxB = jnp.concatenate([receivers[EH:], senders[EH:]])

    xgA = _sc_gather_h(x, idxA)
    xgB = _sc_gather_h(x, idxB)

    def half_edges(xg_h, off):
        return _edge_mlp(edge_attr, xg_h, We1, r2(be1), We2, r2(be2),
                         We3, r2(be3), r2(lne_g), r2(lne_b),
                         ne=EH, block=1280, ea_off=off)

    msgA, eoutA = half_edges(xgA, 0)
    msgB, eoutB = half_edges(xgB, EH // 1280)

    aggA = _sc_scatter_h(msgA, idxA, zeros_nd)
    aggB = _sc_scatter_h(msgB, idxB, zeros_nd)

    x_out = _node_mlp(x, aggA, aggB, Wn1, r2(bn1), Wn2, r2(bn2), Wn3, r2(bn3),
                      r2(lnn_g), r2(lnn_b), block=1000)

    edge_out = jnp.concatenate([eoutA, eoutB], axis=0)
    return x_out, edge_out
